# Initial kernel scaffold; baseline (speedup 1.0000x reference)
import jax
import jax.numpy as jnp
from jax.experimental import pallas as pl


def kernel(x):
    raise NotImplementedError("write your pallas kernel here")


# speedup vs baseline: 3.3984x; 3.3984x over previous
import jax
import jax.numpy as jnp
from jax.experimental import pallas as pl
from jax.experimental.pallas import tpu as pltpu

_ROWS_PER_BLOCK = 512


def _copy_body(x_ref, o_ref):
    o_ref[...] = x_ref[...]


def kernel(x):
    B, C, N = x.shape
    total_rows = B * C
    x2 = x.reshape(total_rows, N)
    grid = total_rows // _ROWS_PER_BLOCK
    out = pl.pallas_call(
        _copy_body,
        grid=(grid,),
        in_specs=[pl.BlockSpec((_ROWS_PER_BLOCK, N), lambda i: (i, 0))],
        out_specs=pl.BlockSpec((_ROWS_PER_BLOCK, N), lambda i: (i, 0)),
        out_shape=jax.ShapeDtypeStruct((total_rows, N), x.dtype),
        compiler_params=pltpu.CompilerParams(
            dimension_semantics=("arbitrary",),
        ),
    )(x2)
    return out.reshape(B, C * 4, N // 4)
